# Initial kernel scaffold; baseline (speedup 1.0000x reference)
import jax
import jax.numpy as jnp
from jax.experimental import pallas as pl


def kernel(x, edge_index, batch, mp_params, cls_params):
    raise NotImplementedError("write your pallas kernel here")


# speedup vs baseline: 1.0981x; 1.0981x over previous
import functools

import jax
import jax.numpy as jnp
from jax.experimental import pallas as pl
from jax.experimental.pallas import tpu as pltpu

_LANES = 128
_ROWS = 1000


def _elu(v):
    return jnp.where(v > 0, v, jnp.exp(jnp.where(v > 0, 0.0, v)) - 1.0)


def _rb(v):
    return v.astype(jnp.bfloat16).astype(jnp.float32)


def _edge_mlp_kernel(xi_ref, xj_ref, w1_ref, b1_ref, w2_ref, b2_ref,
                     w3_ref, b3_ref, out_ref):
    w1 = lambda c, k: w1_ref[c, k].astype(jnp.float32)
    w2 = lambda c, k: w2_ref[c, k].astype(jnp.float32)
    w3 = lambda c, k: w3_ref[c, k].astype(jnp.float32)
    xi = [xi_ref[a] for a in range(3)]
    xj = [xj_ref[a] for a in range(3)]
    corr = [_rb(xi[a] * xj[b]) for a in range(3) for b in range(3)]
    h1 = []
    for k in range(16):
        acc = corr[0] * w1(0, k)
        for c in range(1, 9):
            acc = acc + corr[c] * w1(c, k)
        h1.append(_rb(_elu(acc + b1_ref[0, k])))
    h2 = []
    for k in range(8):
        acc = h1[0] * w2(0, k)
        for i in range(1, 16):
            acc = acc + h1[i] * w2(i, k)
        h2.append(_rb(_elu(acc + b2_ref[0, k])))
    for k in range(3):
        acc = h2[0] * w3(0, k)
        for i in range(1, 8):
            acc = acc + h2[i] * w3(i, k)
        out_ref[k] = acc + b3_ref[0, k]


def _edge_mlp(xi, xj, params):
    (w1, b1), (w2, b2), (w3, b3) = params
    e = xi.shape[1]
    rows = e // _LANES
    xi3 = xi.reshape(3, rows, _LANES)
    xj3 = xj.reshape(3, rows, _LANES)
    grid = rows // _ROWS
    smem = lambda shp: pl.BlockSpec(shp, lambda i: (0,) * len(shp),
                                    memory_space=pltpu.SMEM)
    out = pl.pallas_call(
        _edge_mlp_kernel,
        grid=(grid,),
        in_specs=[
            pl.BlockSpec((3, _ROWS, _LANES), lambda i: (0, i, 0)),
            pl.BlockSpec((3, _ROWS, _LANES), lambda i: (0, i, 0)),
            smem((9, 16)), smem((1, 16)),
            smem((16, 8)), smem((1, 8)),
            smem((8, 3)), smem((1, 3)),
        ],
        out_specs=pl.BlockSpec((3, _ROWS, _LANES), lambda i: (0, i, 0)),
        out_shape=jax.ShapeDtypeStruct((3, rows, _LANES), jnp.float32),
    )(xi3, xj3, w1.astype(jnp.bfloat16), b1.reshape(1, -1),
      w2.astype(jnp.bfloat16), b2.reshape(1, -1),
      w3.astype(jnp.bfloat16), b3.reshape(1, -1))
    return out.reshape(3, e)


def kernel(x, edge_index, batch, mp_params, cls_params):
    n = x.shape[0]
    g = 1024
    src = edge_index[0]
    dst = edge_index[1]
    h = x
    for params in mp_params:
        ht = h.T
        xi = ht[:, dst]
        xj = ht[:, src]
        msg = _edge_mlp(xi, xj, params)
        agg = jax.ops.segment_sum(msg.T, dst, num_segments=n)
        h = h + agg
    ones = jnp.ones((n, 1), dtype=h.dtype)
    counts = jax.ops.segment_sum(ones, batch, num_segments=g)
    pooled = jax.ops.segment_sum(h, batch, num_segments=g) / jnp.maximum(counts, 1.0)
    hcls = pooled
    for i, (w, b) in enumerate(cls_params):
        hcls = hcls @ w + b
        if i < len(cls_params) - 1:
            hcls = _elu(hcls)
    return hcls


# speedup vs baseline: 3.0106x; 2.7416x over previous
import functools

import jax
import jax.numpy as jnp
from jax import lax
from jax.experimental import pallas as pl
from jax.experimental.pallas import tpu as pltpu
from jax.experimental.pallas import tpu_sc as plsc

_LANES = 128
_ROWS = 1000

_NW = 32
_GCHUNK = 5000


def _sc_gather(table, dst, src):
    e = dst.shape[0]
    epw = e // _NW
    nchunk = epw // _GCHUNK
    mesh = plsc.VectorSubcoreMesh(core_axis_name="c", subcore_axis_name="s")

    @functools.partial(
        pl.kernel, mesh=mesh,
        compiler_params=pltpu.CompilerParams(use_tc_tiling_on_sc=False),
        out_type=[jax.ShapeDtypeStruct((e, 16), jnp.float32),
                  jax.ShapeDtypeStruct((e, 16), jnp.float32)],
        scratch_types=[pltpu.VMEM((_GCHUNK,), jnp.int32),
                       pltpu.VMEM((_GCHUNK, 16), jnp.float32),
                       pltpu.SemaphoreType.DMA],
    )
    def gk(table_hbm, dst_hbm, src_hbm, xi_hbm, xj_hbm, idx_v, rows_v, sem):
        wid = lax.axis_index("s") * 2 + lax.axis_index("c")
        base = wid * epw

        def body(c, carry):
            off = base + c * _GCHUNK
            pltpu.sync_copy(dst_hbm.at[pl.ds(off, _GCHUNK)], idx_v)
            pltpu.async_copy(table_hbm.at[idx_v], rows_v, sem).wait()
            pltpu.sync_copy(rows_v, xi_hbm.at[pl.ds(off, _GCHUNK)])
            pltpu.sync_copy(src_hbm.at[pl.ds(off, _GCHUNK)], idx_v)
            pltpu.async_copy(table_hbm.at[idx_v], rows_v, sem).wait()
            pltpu.sync_copy(rows_v, xj_hbm.at[pl.ds(off, _GCHUNK)])
            return carry

        lax.fori_loop(0, nchunk, body, 0)

    return gk(table, dst, src)


def _elu(v):
    return jnp.where(v > 0, v, jnp.exp(jnp.where(v > 0, 0.0, v)) - 1.0)


def _rb(v):
    return v.astype(jnp.bfloat16).astype(jnp.float32)


def _edge_mlp_kernel(xi_ref, xj_ref, w1_ref, b1_ref, w2_ref, b2_ref,
                     w3_ref, b3_ref, out_ref):
    w1 = lambda c, k: w1_ref[c, k].astype(jnp.float32)
    w2 = lambda c, k: w2_ref[c, k].astype(jnp.float32)
    w3 = lambda c, k: w3_ref[c, k].astype(jnp.float32)
    xi = [xi_ref[a] for a in range(3)]
    xj = [xj_ref[a] for a in range(3)]
    corr = [_rb(xi[a] * xj[b]) for a in range(3) for b in range(3)]
    h1 = []
    for k in range(16):
        acc = corr[0] * w1(0, k)
        for c in range(1, 9):
            acc = acc + corr[c] * w1(c, k)
        h1.append(_rb(_elu(acc + b1_ref[0, k])))
    h2 = []
    for k in range(8):
        acc = h1[0] * w2(0, k)
        for i in range(1, 16):
            acc = acc + h1[i] * w2(i, k)
        h2.append(_rb(_elu(acc + b2_ref[0, k])))
    for k in range(3):
        acc = h2[0] * w3(0, k)
        for i in range(1, 8):
            acc = acc + h2[i] * w3(i, k)
        out_ref[k] = acc + b3_ref[0, k]


def _edge_mlp(xi, xj, params):
    (w1, b1), (w2, b2), (w3, b3) = params
    e = xi.shape[1]
    rows = e // _LANES
    xi3 = xi.reshape(3, rows, _LANES)
    xj3 = xj.reshape(3, rows, _LANES)
    grid = rows // _ROWS
    smem = lambda shp: pl.BlockSpec(shp, lambda i: (0,) * len(shp),
                                    memory_space=pltpu.SMEM)
    out = pl.pallas_call(
        _edge_mlp_kernel,
        grid=(grid,),
        in_specs=[
            pl.BlockSpec((3, _ROWS, _LANES), lambda i: (0, i, 0)),
            pl.BlockSpec((3, _ROWS, _LANES), lambda i: (0, i, 0)),
            smem((9, 16)), smem((1, 16)),
            smem((16, 8)), smem((1, 8)),
            smem((8, 3)), smem((1, 3)),
        ],
        out_specs=pl.BlockSpec((3, _ROWS, _LANES), lambda i: (0, i, 0)),
        out_shape=jax.ShapeDtypeStruct((3, rows, _LANES), jnp.float32),
    )(xi3, xj3, w1.astype(jnp.bfloat16), b1.reshape(1, -1),
      w2.astype(jnp.bfloat16), b2.reshape(1, -1),
      w3.astype(jnp.bfloat16), b3.reshape(1, -1))
    return out.reshape(3, e)


def kernel(x, edge_index, batch, mp_params, cls_params):
    n = x.shape[0]
    g = 1024
    src = edge_index[0]
    dst = edge_index[1]
    h = x
    for params in mp_params:
        hp = jnp.zeros((n, 16), jnp.float32).at[:, :3].set(h)
        xi_p, xj_p = _sc_gather(hp, dst, src)
        xi = xi_p[:, :3].T
        xj = xj_p[:, :3].T
        msg = _edge_mlp(xi, xj, params)
        agg = jax.ops.segment_sum(msg.T, dst, num_segments=n)
        h = h + agg
    ones = jnp.ones((n, 1), dtype=h.dtype)
    counts = jax.ops.segment_sum(ones, batch, num_segments=g)
    pooled = jax.ops.segment_sum(h, batch, num_segments=g) / jnp.maximum(counts, 1.0)
    hcls = pooled
    for i, (w, b) in enumerate(cls_params):
        hcls = hcls @ w + b
        if i < len(cls_params) - 1:
            hcls = _elu(hcls)
    return hcls
